# baseline (speedup 1.0000x reference)
import functools

import jax
import jax.numpy as jnp
from jax import lax
from jax.experimental import pallas as pl
from jax.experimental.pallas import tpu as pltpu
from jax.experimental.pallas import tpu_sc as plsc

_D = 100
_P = 128
_NC = 2
_NS = 16
_NW = _NC * _NS
_ROWS = 200
_DIRECT = False
_NBUF = 4 if _DIRECT else 2


def _repack(src, dst):
    for r in range(_ROWS):
        for k in range(_D // 16):
            dst[r, pl.ds(16 * k, 16)] = src[r, pl.ds(16 * k, 16)]
        dst[r, pl.ds(_D - 16, 16)] = src[r, pl.ds(_D - 16, 16)]


def _gather_sc(xf, tpad, nrows_total):
    nb = nrows_total // (_NW * _ROWS)
    bpt = nb * _ROWS
    mesh = plsc.VectorSubcoreMesh(core_axis_name="c", subcore_axis_name="s")

    if _DIRECT:
        scratch = (
            [pltpu.VMEM((bpt,), jnp.int32)]
            + [pltpu.VMEM((_ROWS, _D), jnp.float32) for _ in range(_NBUF)]
            + [pltpu.SemaphoreType.DMA for _ in range(2 * _NBUF)]
        )
    else:
        scratch = (
            [pltpu.VMEM((bpt,), jnp.int32)]
            + [pltpu.VMEM((_ROWS, _P), jnp.float32) for _ in range(_NBUF)]
            + [pltpu.VMEM((_ROWS, _D), jnp.float32) for _ in range(_NBUF)]
            + [pltpu.SemaphoreType.DMA for _ in range(2 * _NBUF)]
        )

    @functools.partial(
        pl.kernel,
        out_type=jax.ShapeDtypeStruct((_NW * nb, _ROWS, _D), jnp.float32),
        mesh=mesh,
        scratch_types=scratch,
        compiler_params=pltpu.CompilerParams(use_tc_tiling_on_sc=True),
    )
    def k(x_hbm, tbl_hbm, out_hbm, idx_v, *rest):
        if _DIRECT:
            cbufs = rest[:_NBUF]
            gbufs = cbufs
            sems = rest[_NBUF:]
        else:
            gbufs = rest[:_NBUF]
            cbufs = rest[_NBUF:2 * _NBUF]
            sems = rest[2 * _NBUF:]
        gsem = sems[:_NBUF]
        osem = sems[_NBUF:]
        wid = lax.axis_index("s") * _NC + lax.axis_index("c")
        pltpu.sync_copy(x_hbm.at[pl.ds(wid * bpt, bpt)], idx_v)

        def start_gather(i, b):
            off = i * _ROWS
            pltpu.async_copy(
                tbl_hbm.at[idx_v.at[pl.ds(off, 128)]],
                gbufs[b].at[pl.ds(0, 128)], gsem[b])
            pltpu.async_copy(
                tbl_hbm.at[idx_v.at[pl.ds(off + 128, _ROWS - 128)]],
                gbufs[b].at[pl.ds(128, _ROWS - 128)], gsem[b])

        def wait_gather(i, b):
            off = i * _ROWS
            pltpu.make_async_copy(
                tbl_hbm.at[idx_v.at[pl.ds(off, 128)]],
                gbufs[b].at[pl.ds(0, 128)], gsem[b]).wait()
            pltpu.make_async_copy(
                tbl_hbm.at[idx_v.at[pl.ds(off + 128, _ROWS - 128)]],
                gbufs[b].at[pl.ds(128, _ROWS - 128)], gsem[b]).wait()

        def wait_out(b):
            pltpu.make_async_copy(
                cbufs[b], out_hbm.at[wid * nb], osem[b]).wait()

        for b in range(_NBUF):
            start_gather(b, b)

        @pl.loop(0, nb, step=_NBUF)
        def _(i0):
            for b in range(_NBUF):
                i = i0 + b
                wait_gather(i, b)

                if not _DIRECT:
                    @pl.when(i >= _NBUF)
                    def _():
                        wait_out(b)
                    _repack(gbufs[b], cbufs[b])

                pltpu.async_copy(cbufs[b], out_hbm.at[wid * nb + i], osem[b])

                @pl.when(i + _NBUF < nb)
                def _():
                    if _DIRECT:
                        wait_out(b)
                    start_gather(i + _NBUF, b)

        for b in range(_NBUF):
            wait_out(b)

    return k(xf, tpad)


def kernel(X, table):
    n, m = X.shape
    total = n * m
    xf = X.reshape(total).astype(jnp.int32)
    tpad = jnp.pad(table.astype(jnp.float32), ((0, 0), (0, _P - _D)))
    out = _gather_sc(xf, tpad, total)
    return out.reshape(n, m, _D)
